# Initial kernel scaffold; baseline (speedup 1.0000x reference)
import jax
import jax.numpy as jnp
from jax.experimental import pallas as pl


def kernel(labels, table, force_drop_ids):
    raise NotImplementedError("write your pallas kernel here")


# speedup vs baseline: 2.5422x; 2.5422x over previous
import functools

import jax
import jax.numpy as jnp
from jax.experimental import pallas as pl
from jax.experimental.pallas import tpu as pltpu


def _gather_tile_kernel(lbl_ref, drop_ref, table_ref, out_ref, sem,
                        *, tile_b: int, cfg_row: int):
    base = pl.program_id(0) * tile_b
    for r in range(tile_b):
        lbl = lbl_ref[base + r]
        drop = drop_ref[base + r]
        row = jnp.where(drop == 1, cfg_row, lbl)
        row = jnp.clip(row, 0, cfg_row)
        pltpu.make_async_copy(table_ref.at[row], out_ref.at[r], sem).start()
    pltpu.make_async_copy(
        table_ref.at[pl.ds(0, tile_b)], out_ref.at[pl.ds(0, tile_b)], sem
    ).wait()


def kernel(labels, table, force_drop_ids):
    (B,) = labels.shape
    V, H = table.shape
    cfg_row = V - 1

    labels = labels.astype(jnp.int32)
    force_drop_ids = force_drop_ids.astype(jnp.int32)

    tile_b = 256
    while B % tile_b != 0:
        tile_b //= 2
    n_b = B // tile_b

    table3 = table.reshape(V, 1, H)
    itemsize = jnp.dtype(table.dtype).itemsize

    grid_spec = pltpu.PrefetchScalarGridSpec(
        num_scalar_prefetch=2,
        grid=(n_b,),
        in_specs=[pl.BlockSpec(memory_space=pl.ANY)],
        out_specs=pl.BlockSpec((tile_b, 1, H), lambda i, lbl, drp: (i, 0, 0)),
        scratch_shapes=[pltpu.SemaphoreType.DMA],
    )
    out = pl.pallas_call(
        functools.partial(_gather_tile_kernel, tile_b=tile_b, cfg_row=cfg_row),
        out_shape=jax.ShapeDtypeStruct((B, 1, H), table.dtype),
        grid_spec=grid_spec,
        compiler_params=pltpu.CompilerParams(
            dimension_semantics=("parallel",),
            disable_bounds_checks=True,
        ),
        cost_estimate=pl.CostEstimate(
            flops=0,
            transcendentals=0,
            bytes_accessed=2 * B * H * itemsize + 8 * B),
    )(labels, force_drop_ids, table3)
    return out.reshape(B, H)


# speedup vs baseline: 2.5450x; 1.0011x over previous
import functools

import jax
import jax.numpy as jnp
from jax.experimental import pallas as pl
from jax.experimental.pallas import tpu as pltpu


def _gather_tile_kernel(lbl_ref, drop_ref, table_ref, out_ref, sems,
                        *, tile_b: int, cfg_row: int, n_sem: int):
    base = pl.program_id(0) * tile_b
    for r in range(tile_b):
        lbl = lbl_ref[base + r]
        drop = drop_ref[base + r]
        row = jnp.where(drop == 1, cfg_row, lbl)
        row = jnp.clip(row, 0, cfg_row)
        pltpu.make_async_copy(
            table_ref.at[row], out_ref.at[r], sems.at[r % n_sem]).start()
    per_sem = tile_b // n_sem
    for j in range(n_sem):
        pltpu.make_async_copy(
            table_ref.at[pl.ds(0, per_sem)], out_ref.at[pl.ds(0, per_sem)],
            sems.at[j],
        ).wait()


def kernel(labels, table, force_drop_ids):
    (B,) = labels.shape
    V, H = table.shape
    cfg_row = V - 1

    labels = labels.astype(jnp.int32)
    force_drop_ids = force_drop_ids.astype(jnp.int32)

    tile_b = 256
    while B % tile_b != 0:
        tile_b //= 2
    n_b = B // tile_b
    n_sem = 8
    while tile_b % n_sem != 0:
        n_sem //= 2

    table3 = table.reshape(V, 1, H)
    itemsize = jnp.dtype(table.dtype).itemsize

    grid_spec = pltpu.PrefetchScalarGridSpec(
        num_scalar_prefetch=2,
        grid=(n_b,),
        in_specs=[pl.BlockSpec(memory_space=pl.ANY)],
        out_specs=pl.BlockSpec((tile_b, 1, H), lambda i, lbl, drp: (i, 0, 0)),
        scratch_shapes=[pltpu.SemaphoreType.DMA((n_sem,))],
    )
    out = pl.pallas_call(
        functools.partial(_gather_tile_kernel, tile_b=tile_b, cfg_row=cfg_row,
                          n_sem=n_sem),
        out_shape=jax.ShapeDtypeStruct((B, 1, H), table.dtype),
        grid_spec=grid_spec,
        compiler_params=pltpu.CompilerParams(
            dimension_semantics=("parallel",),
            disable_bounds_checks=True,
        ),
        cost_estimate=pl.CostEstimate(
            flops=0,
            transcendentals=0,
            bytes_accessed=2 * B * H * itemsize + 8 * B),
    )(labels, force_drop_ids, table3)
    return out.reshape(B, H)
